# baseline (speedup 1.0000x reference)
import functools

import jax
import jax.numpy as jnp
from jax import lax
from jax.experimental import pallas as pl
from jax.experimental.pallas import tpu as pltpu
from jax.experimental.pallas import tpu_sc as plsc

N = 100000
D = 128
G = 4096
H_OUT = 12

NC = 2
NS = 16
NW = NC * NS
CHUNK = 128
NFULL = N // CHUNK
TAIL = N - NFULL * CHUNK
JMAX = 25
ROWS_PER_SID = G // NS
NV = D // 16
GARBAGE = G
GACC = G + CHUNK


def _sc_segment_sum(node_feats, idx_t, zrow):
    mesh = plsc.VectorSubcoreMesh(
        core_axis_name="c", subcore_axis_name="s", num_cores=NC, num_subcores=NS
    )

    @functools.partial(
        pl.kernel,
        out_type=jax.ShapeDtypeStruct((NC, G, D), jnp.float32),
        mesh=mesh,
        scratch_types=[
            pltpu.VMEM((JMAX, CHUNK), jnp.int32),
            pltpu.VMEM((2, CHUNK, D), jnp.float32),
            pltpu.VMEM((CHUNK, D), jnp.float32),
            pltpu.VMEM((CHUNK, D), jnp.float32),
            pltpu.VMEM((1, CHUNK), jnp.int32),
            pltpu.VMEM((16,), jnp.int32),
            pltpu.VMEM((32,), jnp.int32),
            pltpu.VMEM((64,), jnp.int32),
            pltpu.VMEM_SHARED((GACC, D), jnp.float32),
            pltpu.SemaphoreType.DMA((2,)),
        ],
    )
    def seg_sum(node_hbm, idx_hbm, zrow_hbm, out_hbm, ids_buf, rbuf, zbuf,
                cbuf, idsC, ids16, ids32, ids64, acc, sem):
        cid = lax.axis_index("c")
        sid = lax.axis_index("s")
        wid = cid * NS + sid

        pltpu.sync_copy(idx_hbm.at[wid], ids_buf)
        pltpu.sync_copy(zrow_hbm, zbuf)

        base = sid * ROWS_PER_SID
        pltpu.sync_copy(zbuf, acc.at[pl.ds(base, CHUNK)])
        pltpu.sync_copy(zbuf, acc.at[pl.ds(base + CHUNK, CHUNK)])
        plsc.subcore_barrier()

        nfull = jnp.where(wid <= 12, JMAX, JMAX - 1)

        pltpu.async_copy(
            node_hbm.at[pl.ds(wid * CHUNK, CHUNK)], rbuf.at[0], sem.at[0]
        )

        gvec = jnp.full((16,), GARBAGE, jnp.int32)
        zv = jnp.zeros((16,), jnp.float32)

        def body(j, carry):
            b = lax.rem(j, 2)
            nb = 1 - b

            @pl.when(j + 1 < nfull)
            def _():
                c1 = wid + NW * (j + 1)
                pltpu.async_copy(
                    node_hbm.at[pl.ds(c1 * CHUNK, CHUNK)], rbuf.at[nb], sem.at[nb]
                )

            pltpu.make_async_copy(
                node_hbm.at[pl.ds(0, CHUNK)], rbuf.at[b], sem.at[b]
            ).wait()

            lane = lax.iota(jnp.int32, 16)

            def row_step(i, idv, c):
                avec = list(c[0:NV])
                kvec = list(c[NV:2 * NV])
                prev = c[2 * NV]
                k = c[2 * NV + 1]
                same = idv == prev
                knew = jnp.where(same, k, k + 1)
                navec = [
                    jnp.where(same, avec[v] + rbuf[b, i, pl.ds(v * 16, 16)],
                              rbuf[b, i, pl.ds(v * 16, 16)])
                    for v in range(NV)
                ]
                for v in range(NV):
                    cbuf[knew, pl.ds(v * 16, 16)] = navec[v]
                nkvec = [
                    jnp.where(lane + 16 * v == knew, idv, kvec[v])
                    for v in range(NV)
                ]
                return (*navec, *nkvec, idv, knew)

            def group(g, c):
                ids_v = ids_buf[j, pl.ds(g * 16, 16)]
                for r in range(16):
                    c = row_step(g * 16 + r, ids_v[r], c)
                return c

            init = (zv,) * NV + (gvec,) * NV + (jnp.int32(-1), jnp.int32(-1))
            fin = lax.fori_loop(0, CHUNK // 16, group, init)
            k = fin[2 * NV + 1]
            for v in range(NV):
                idsC[0, pl.ds(v * 16, 16)] = fin[NV + v]
            kcnt = k + 1

            @pl.when(kcnt <= 16)
            def _():
                ids16[pl.ds(0, 16)] = idsC[0, pl.ds(0, 16)]
                pltpu.sync_copy(cbuf.at[pl.ds(0, 16)], acc.at[ids16], add=True)

            @pl.when(jnp.logical_and(kcnt > 16, kcnt <= 32))
            def _():
                for v in range(2):
                    ids32[pl.ds(v * 16, 16)] = idsC[0, pl.ds(v * 16, 16)]
                pltpu.sync_copy(cbuf.at[pl.ds(0, 32)], acc.at[ids32], add=True)

            @pl.when(jnp.logical_and(kcnt > 32, kcnt <= 64))
            def _():
                for v in range(4):
                    ids64[pl.ds(v * 16, 16)] = idsC[0, pl.ds(v * 16, 16)]
                pltpu.sync_copy(cbuf.at[pl.ds(0, 64)], acc.at[ids64], add=True)

            @pl.when(kcnt > 64)
            def _():
                pltpu.sync_copy(cbuf, acc.at[idsC.at[0]], add=True)

            return carry

        lax.fori_loop(0, nfull, body, 0)

        @pl.when(wid == 13)
        def _():
            pltpu.sync_copy(
                node_hbm.at[pl.ds(NFULL * CHUNK, TAIL)], zbuf.at[pl.ds(0, TAIL)]
            )
            pltpu.sync_copy(zbuf, acc.at[ids_buf.at[JMAX - 1]], add=True)

        plsc.subcore_barrier()

        pltpu.sync_copy(
            acc.at[pl.ds(base, ROWS_PER_SID)],
            out_hbm.at[cid, pl.ds(base, ROWS_PER_SID)],
        )

    return seg_sum(node_feats, idx_t, zrow)


def _tc_body(p_ref, g_ref, b_ref, w1_ref, b1_ref, w2_ref, b2_ref, o_ref):
    x = p_ref[0] + p_ref[1]
    mean = jnp.mean(x, axis=0, keepdims=True)
    xc = x - mean
    var = jnp.mean(xc * xc, axis=0, keepdims=True)
    gn = xc * lax.rsqrt(var + 1e-5) * g_ref[...] + b_ref[...]
    h = jnp.dot(gn, w1_ref[...], preferred_element_type=jnp.float32) + b1_ref[...]
    h = jnp.maximum(h, 0.0)
    o_ref[...] = jnp.dot(h, w2_ref[...], preferred_element_type=jnp.float32) + b2_ref[...]


def _tc_bn_mlp(partials, gamma, beta, W1, b1, W2, b2):
    return pl.pallas_call(
        _tc_body,
        out_shape=jax.ShapeDtypeStruct((G, H_OUT), jnp.float32),
    )(partials, gamma, beta, W1, b1, W2, b2)


def kernel(node_feats, segment_ids, gamma, beta, W1, b1, W2, b2):
    ids_i32 = segment_ids.astype(jnp.int32)
    ids_pad = jnp.zeros((NW * JMAX * CHUNK,), jnp.int32).at[:N].set(ids_i32)
    idx_t = ids_pad.reshape(JMAX, NW, CHUNK).transpose(1, 0, 2)
    zrow = jnp.zeros((CHUNK, D), jnp.float32)

    partials = _sc_segment_sum(node_feats, idx_t, zrow)

    return _tc_bn_mlp(
        partials,
        gamma.reshape(1, D),
        beta.reshape(1, D),
        W1,
        b1.reshape(1, D),
        W2,
        b2.reshape(1, H_OUT),
    )
